# baseline (speedup 1.0000x reference)
import functools

import jax
import jax.numpy as jnp
from jax import lax
from jax.experimental import pallas as pl
from jax.experimental.pallas import tpu as pltpu
from jax.experimental.pallas import tpu_sc as plsc

NUM_CORES = 2
NUM_SUBCORES = 16

W = 512
K = W // 128
NH = 200
NHH = NH // 8
NBUF = 3


@jax.jit
def _sc_gather(weight, idx4):
    mesh = plsc.VectorSubcoreMesh(
        core_axis_name="c", subcore_axis_name="s",
        num_cores=NUM_CORES, num_subcores=NUM_SUBCORES,
    )

    @functools.partial(
        pl.kernel,
        out_type=jax.ShapeDtypeStruct((NH, 4, 128, 8, 128), jnp.float32),
        mesh=mesh,
        scratch_types=[
            pltpu.VMEM((2, K, 8, 128), jnp.int32),
            pltpu.VMEM((NBUF, W, 32), jnp.float32),
            pltpu.VMEM((NBUF, 4, K, 8, 128), jnp.float32),
            pltpu.SemaphoreType.DMA((NBUF,)),
            pltpu.SemaphoreType.DMA((NBUF,)),
            pltpu.SemaphoreType.DMA,
        ],
        compiler_params=pltpu.CompilerParams(
            use_tc_tiling_on_sc=False, needs_layout_passes=False),
    )
    def k(table_hbm, idx_hbm, out_hbm, idxg_v, rows_v, trans_v,
          gsems, osems, isem):
        wid = lax.axis_index("s") * NUM_CORES + lax.axis_index("c")
        bh0 = wid * K
        iota = lax.iota(jnp.int32, 16)

        def idx_group_copy(hh, g):
            return pltpu.make_async_copy(
                idx_hbm.at[hh, pl.ds(bh0, K)], idxg_v.at[g], isem)

        def gathers(hl, g, s):
            return [pltpu.make_async_copy(
                        table_hbm.at[idxg_v.at[g, j, hl]],
                        rows_v.at[s, pl.ds(j * 128, 128)],
                        gsems.at[s])
                    for j in range(K)]

        def stores(h, s):
            return [pltpu.make_async_copy(
                        trans_v.at[s, dh],
                        out_hbm.at[h, dh, pl.ds(bh0, K)], osems.at[s])
                    for dh in range(4)]

        idx_group_copy(0, 0).start()
        idx_group_copy(0, 0).wait()
        for t0 in range(NBUF):
            for d in gathers(t0, 0, t0):
                d.start()

        @pl.loop(0, NH)
        def _unit(t):
            hh = t // 8
            hl = lax.rem(t, 8)
            g = lax.rem(hh, 2)
            s = lax.rem(t, NBUF)

            @pl.when((hl == 0) & (hh < NHH - 1))
            def _prefetch():
                idx_group_copy(hh + 1, 1 - g).start()

            for d in gathers(hl, g, s):
                d.wait()

            @pl.when(t >= NBUF)
            def _drain():
                for d in stores(t - NBUF, s):
                    d.wait()

            @plsc.parallel_loop(0, 4 * K)
            def _tp(j):
                dh = j // K
                kk = lax.rem(j, K)
                for dl in range(8):
                    col = jnp.full((16,), dh * 8 + dl, jnp.int32)
                    for b16 in range(8):
                        row = iota + (kk * 128 + b16 * 16)
                        v = plsc.load_gather(rows_v.at[s], [row, col])
                        trans_v[s, dh, kk, dl, pl.ds(b16 * 16, 16)] = v

            for d in stores(t, s):
                d.start()

            @pl.when((hl == 8 - NBUF) & (hh < NHH - 1))
            def _iwait():
                idx_group_copy(hh + 1, 1 - g).wait()

            @pl.when(t + NBUF < NH)
            def _refill():
                t2 = t + NBUF
                for d in gathers(lax.rem(t2, 8), lax.rem(t2 // 8, 2), s):
                    d.start()

        for t0 in range(NH - NBUF, NH):
            for d in stores(t0, t0 % NBUF):
                d.wait()

    return k(weight, idx4)


def kernel(indices, weight):
    idx4 = indices.astype(jnp.int32).reshape(128, 128, NHH, 8)
    idx4 = idx4.transpose(2, 0, 3, 1)
    out5 = _sc_gather(weight, idx4)
    out = out5.transpose(2, 4, 0, 1, 3).reshape(16384, NH, 32)
    return out


# speedup vs baseline: 1.5271x; 1.5271x over previous
import functools

import jax
import jax.numpy as jnp
from jax import lax
from jax.experimental import pallas as pl
from jax.experimental.pallas import tpu as pltpu
from jax.experimental.pallas import tpu_sc as plsc

NUM_CORES = 2
NUM_SUBCORES = 16

W = 512
K = W // 128
NH = 200
NHH = NH // 8
NBUF = 3


@jax.jit
def _sc_gather(weight, idx4):
    mesh = plsc.VectorSubcoreMesh(
        core_axis_name="c", subcore_axis_name="s",
        num_cores=NUM_CORES, num_subcores=NUM_SUBCORES,
    )

    @functools.partial(
        pl.kernel,
        out_type=jax.ShapeDtypeStruct((NH, 4, 128, 8, 128), jnp.float32),
        mesh=mesh,
        scratch_types=[
            pltpu.VMEM((2, K, 8, 128), jnp.int32),
            pltpu.VMEM((NBUF, W, 32), jnp.float32),
            pltpu.VMEM((W, 33), jnp.float32),
            pltpu.VMEM((NBUF, 4, K, 8, 128), jnp.float32),
            pltpu.SemaphoreType.DMA((NBUF,)),
            pltpu.SemaphoreType.DMA((NBUF,)),
            pltpu.SemaphoreType.DMA,
        ],
        compiler_params=pltpu.CompilerParams(
            use_tc_tiling_on_sc=False, needs_layout_passes=False),
    )
    def k(table_hbm, idx_hbm, out_hbm, idxg_v, rows_v, pad_v, trans_v,
          gsems, osems, isem):
        wid = lax.axis_index("s") * NUM_CORES + lax.axis_index("c")
        bh0 = wid * K
        iota = lax.iota(jnp.int32, 16)

        def idx_group_copy(hh, g):
            return pltpu.make_async_copy(
                idx_hbm.at[hh, pl.ds(bh0, K)], idxg_v.at[g], isem)

        def gathers(hl, g, s):
            return [pltpu.make_async_copy(
                        table_hbm.at[idxg_v.at[g, j, hl]],
                        rows_v.at[s, pl.ds(j * 128, 128)],
                        gsems.at[s])
                    for j in range(K)]

        def stores(h, s):
            return [pltpu.make_async_copy(
                        trans_v.at[s, dh],
                        out_hbm.at[h, dh, pl.ds(bh0, K)], osems.at[s])
                    for dh in range(4)]

        idx_group_copy(0, 0).start()
        idx_group_copy(0, 0).wait()
        for t0 in range(NBUF):
            for d in gathers(t0, 0, t0):
                d.start()

        @pl.loop(0, NH)
        def _unit(t):
            hh = t // 8
            hl = lax.rem(t, 8)
            g = lax.rem(hh, 2)
            s = lax.rem(t, NBUF)

            @pl.when((hl == 0) & (hh < NHH - 1))
            def _prefetch():
                idx_group_copy(hh + 1, 1 - g).start()

            for d in gathers(hl, g, s):
                d.wait()

            @plsc.parallel_loop(0, W)
            def _rp(r):
                for c in range(2):
                    v = rows_v[s, r, pl.ds(c * 16, 16)]
                    pad_v[r, pl.ds(c * 16, 16)] = v

            @pl.when(t >= NBUF)
            def _drain():
                for d in stores(t - NBUF, s):
                    d.wait()

            @plsc.parallel_loop(0, 4 * K)
            def _tp(j):
                dh = j // K
                kk = lax.rem(j, K)
                for dl in range(8):
                    col = jnp.full((16,), dh * 8 + dl, jnp.int32)
                    for b16 in range(8):
                        row = iota + (kk * 128 + b16 * 16)
                        v = plsc.load_gather(pad_v, [row, col])
                        trans_v[s, dh, kk, dl, pl.ds(b16 * 16, 16)] = v

            for d in stores(t, s):
                d.start()

            @pl.when((hl == 8 - NBUF) & (hh < NHH - 1))
            def _iwait():
                idx_group_copy(hh + 1, 1 - g).wait()

            @pl.when(t + NBUF < NH)
            def _refill():
                t2 = t + NBUF
                for d in gathers(lax.rem(t2, 8), lax.rem(t2 // 8, 2), s):
                    d.start()

        for t0 in range(NH - NBUF, NH):
            for d in stores(t0, t0 % NBUF):
                d.wait()

    return k(weight, idx4)


def kernel(indices, weight):
    idx4 = indices.astype(jnp.int32).reshape(128, 128, NHH, 8)
    idx4 = idx4.transpose(2, 0, 3, 1)
    out5 = _sc_gather(weight, idx4)
    out = out5.transpose(2, 4, 0, 1, 3).reshape(16384, NH, 32)
    return out
